# baseline (speedup 1.0000x reference)
import functools

import jax
import jax.numpy as jnp
from jax import lax
from jax.experimental import pallas as pl
from jax.experimental.pallas import tpu as pltpu
from jax.experimental.pallas import tpu_sc as plsc

B, N, D = 64, 1024, 768
B_SC = 24
B_TC = B - B_SC
NC, NS, L = 2, 16, 16
NW = NC * NS
RPW = N // NW
U = 8
NBUF = 4
ROUNDS = B_SC // NBUF

_mesh = plsc.VectorSubcoreMesh(core_axis_name="c", subcore_axis_name="s")


@functools.partial(
    pl.kernel,
    mesh=_mesh,
    out_type=jax.ShapeDtypeStruct((B_SC, N, D), jnp.float32),
    scratch_types=(
        [pltpu.VMEM((RPW, D), jnp.float32) for _ in range(NBUF + 1)]
        + [pltpu.SemaphoreType.DMA for _ in range(2 * NBUF)]
    ),
)
def _sc_pos_add(patch_hbm, pos_hbm, out_hbm, *refs):
    ios = list(refs[:NBUF])
    pos_v = refs[NBUF]
    sin = list(refs[NBUF + 1 : NBUF + 1 + NBUF])
    sout = list(refs[NBUF + 1 + NBUF :])

    wid = lax.axis_index("s") * NC + lax.axis_index("c")
    n0 = wid * RPW
    pltpu.sync_copy(pos_hbm.at[pl.ds(n0, RPW)], pos_v)

    for j in range(NBUF):
        pltpu.async_copy(patch_hbm.at[j, pl.ds(n0, RPW)], ios[j], sin[j])

    def add_pos(io):
        def row_body(r, c):
            for k in range(D // (L * U)):
                for u in range(U):
                    sl = pl.ds(k * L * U + u * L, L)
                    plsc.addupdate(io.at[r, sl], pos_v[r, sl])
            return c

        lax.fori_loop(0, RPW, row_body, 0)

    def round_body(g, carry):
        b0 = g * NBUF
        for j in range(NBUF):
            b = b0 + j
            pltpu.make_async_copy(patch_hbm.at[b, pl.ds(n0, RPW)], ios[j], sin[j]).wait()
            add_pos(ios[j])
            pltpu.async_copy(ios[j], out_hbm.at[b, pl.ds(n0, RPW)], sout[j])

        @pl.when(g + 1 < ROUNDS)
        def _():
            for j in range(NBUF):
                pltpu.make_async_copy(ios[j], out_hbm.at[b0, pl.ds(n0, RPW)], sout[j]).wait()
                pltpu.async_copy(patch_hbm.at[b0 + NBUF + j, pl.ds(n0, RPW)], ios[j], sin[j])

        return carry

    lax.fori_loop(0, ROUNDS, round_body, 0)

    for j in range(NBUF):
        pltpu.make_async_copy(ios[j], out_hbm.at[0, pl.ds(n0, RPW)], sout[j]).wait()


def _tc_body(patch_ref, pos_ref, out_ref):
    out_ref[...] = patch_ref[...] + pos_ref[...][None]


_tc_pos_add = pl.pallas_call(
    _tc_body,
    grid=(B_TC,),
    in_specs=[
        pl.BlockSpec((1, N, D), lambda i: (i + B_SC, 0, 0)),
        pl.BlockSpec((N, D), lambda i: (0, 0)),
    ],
    out_specs=pl.BlockSpec((1, N, D), lambda i: (i, 0, 0)),
    out_shape=jax.ShapeDtypeStruct((B_TC, N, D), jnp.float32),
)


def kernel(patch, pos_table):
    sc_out = _sc_pos_add(patch, pos_table)
    tc_out = _tc_pos_add(patch, pos_table)
    return jnp.concatenate([sc_out, tc_out], axis=0)


# speedup vs baseline: 2.1001x; 2.1001x over previous
import functools

import jax
import jax.numpy as jnp
from jax import lax
from jax.experimental import pallas as pl
from jax.experimental.pallas import tpu as pltpu
from jax.experimental.pallas import tpu_sc as plsc

B, N, D = 64, 1024, 768
B_SC = 0
B_TC = B - B_SC
NC, NS, L = 2, 16, 16
NW = NC * NS
RPW = N // NW
U = 8
NBUF = 4
ROUNDS = max(B_SC // NBUF, 1)

_mesh = plsc.VectorSubcoreMesh(core_axis_name="c", subcore_axis_name="s")


@functools.partial(
    pl.kernel,
    mesh=_mesh,
    out_type=jax.ShapeDtypeStruct((max(B_SC,1), N, D), jnp.float32),
    scratch_types=(
        [pltpu.VMEM((RPW, D), jnp.float32) for _ in range(NBUF + 1)]
        + [pltpu.SemaphoreType.DMA for _ in range(2 * NBUF)]
    ),
)
def _sc_pos_add(patch_hbm, pos_hbm, out_hbm, *refs):
    ios = list(refs[:NBUF])
    pos_v = refs[NBUF]
    sin = list(refs[NBUF + 1 : NBUF + 1 + NBUF])
    sout = list(refs[NBUF + 1 + NBUF :])

    wid = lax.axis_index("s") * NC + lax.axis_index("c")
    n0 = wid * RPW
    pltpu.sync_copy(pos_hbm.at[pl.ds(n0, RPW)], pos_v)

    for j in range(NBUF):
        pltpu.async_copy(patch_hbm.at[j, pl.ds(n0, RPW)], ios[j], sin[j])

    def add_pos(io):
        def row_body(r, c):
            for k in range(D // (L * U)):
                for u in range(U):
                    sl = pl.ds(k * L * U + u * L, L)
                    plsc.addupdate(io.at[r, sl], pos_v[r, sl])
            return c

        lax.fori_loop(0, RPW, row_body, 0)

    def round_body(g, carry):
        b0 = g * NBUF
        for j in range(NBUF):
            b = b0 + j
            pltpu.make_async_copy(patch_hbm.at[b, pl.ds(n0, RPW)], ios[j], sin[j]).wait()
            add_pos(ios[j])
            pltpu.async_copy(ios[j], out_hbm.at[b, pl.ds(n0, RPW)], sout[j])

        @pl.when(g + 1 < ROUNDS)
        def _():
            for j in range(NBUF):
                pltpu.make_async_copy(ios[j], out_hbm.at[b0, pl.ds(n0, RPW)], sout[j]).wait()
                pltpu.async_copy(patch_hbm.at[b0 + NBUF + j, pl.ds(n0, RPW)], ios[j], sin[j])

        return carry

    lax.fori_loop(0, ROUNDS, round_body, 0)

    for j in range(NBUF):
        pltpu.make_async_copy(ios[j], out_hbm.at[0, pl.ds(n0, RPW)], sout[j]).wait()


def _tc_body(patch_ref, pos_ref, out_ref):
    out_ref[...] = patch_ref[...] + pos_ref[...][None]


_tc_pos_add = pl.pallas_call(
    _tc_body,
    grid=(B_TC,),
    in_specs=[
        pl.BlockSpec((1, N, D), lambda i: (i + B_SC, 0, 0)),
        pl.BlockSpec((N, D), lambda i: (0, 0)),
    ],
    out_specs=pl.BlockSpec((1, N, D), lambda i: (i, 0, 0)),
    out_shape=jax.ShapeDtypeStruct((B_TC, N, D), jnp.float32),
)


def kernel(patch, pos_table):
    return _tc_pos_add(patch, pos_table)
